# Initial kernel scaffold; baseline (speedup 1.0000x reference)
import jax
import jax.numpy as jnp
from jax.experimental import pallas as pl


def kernel(node_features, edges, W1, b1, W2, b2):
    raise NotImplementedError("write your pallas kernel here")


# speedup vs baseline: 10.7308x; 10.7308x over previous
import functools

import jax
import jax.numpy as jnp
from jax import lax
from jax.experimental import pallas as pl
from jax.experimental.pallas import tpu as pltpu
from jax.experimental.pallas import tpu_sc as plsc

NC = 2
NS = 16
NW = NC * NS
L = 16
CH = 128

N = 10000
D = 128
E = 320000
NPAD = 10240
PT = -(-E // (NW * CH)) * CH
EPAD = PT * NW


def _mesh():
    return plsc.VectorSubcoreMesh(core_axis_name="c", subcore_axis_name="s")


DW = 16


def _deg_kernel(dstp):
    @functools.partial(
        pl.kernel,
        mesh=_mesh(),
        out_type=jax.ShapeDtypeStruct((NC, NPAD, DW), jnp.float32),
        scratch_types=[
            pltpu.VMEM((CH,), jnp.int32),
            pltpu.VMEM((CH, DW), jnp.float32),
            pltpu.VMEM((CH, DW), jnp.float32),
            pltpu.VMEM_SHARED((NPAD, DW), jnp.float32),
        ],
    )
    def k(dst_hbm, out_hbm, dbuf, ones_buf, zbuf, acc_sh):
        c = lax.axis_index("c")
        s = lax.axis_index("s")
        w = c * NS + s

        def fill(i, _):
            ones_buf[i, :] = jnp.ones((DW,), jnp.float32)
            zbuf[i, :] = jnp.zeros((DW,), jnp.float32)
            return 0

        lax.fori_loop(0, CH, fill, 0)

        rows_per_tile = NPAD // NS

        def zcopy(r, _):
            pltpu.sync_copy(zbuf, acc_sh.at[pl.ds(s * rows_per_tile + r * CH, CH)])
            return 0

        lax.fori_loop(0, rows_per_tile // CH, zcopy, 0)
        plsc.subcore_barrier()

        def edge_body(g, _):
            base = w * PT + g * CH
            pltpu.sync_copy(dst_hbm.at[pl.ds(base, CH)], dbuf)
            pltpu.sync_copy(ones_buf, acc_sh.at[dbuf], add=True)
            return 0

        lax.fori_loop(0, PT // CH, edge_body, 0)
        plsc.subcore_barrier()

        pltpu.sync_copy(
            acc_sh.at[pl.ds(s * rows_per_tile, rows_per_tile)],
            out_hbm.at[c, pl.ds(s * rows_per_tile, rows_per_tile)],
        )

    return k(dstp)


def _scatter_kernel(z, srcp, dstp):
    @functools.partial(
        pl.kernel,
        mesh=_mesh(),
        out_type=jax.ShapeDtypeStruct((NC, NPAD, D), jnp.float32),
        scratch_types=[
            pltpu.VMEM((CH,), jnp.int32),
            pltpu.VMEM((CH,), jnp.int32),
            pltpu.VMEM((CH, D), jnp.float32),
            pltpu.VMEM((CH, D), jnp.float32),
            pltpu.VMEM_SHARED((NPAD, D), jnp.float32),
            pltpu.SemaphoreType.DMA,
        ],
    )
    def k(z_hbm, src_hbm, dst_hbm, out_hbm, sbuf, dbuf, rows, zbuf, acc_sh, sem):
        c = lax.axis_index("c")
        s = lax.axis_index("s")
        w = c * NS + s

        def zrow(i, _):
            for j in range(D // L):
                zbuf[i, pl.ds(j * L, L)] = jnp.zeros((L,), jnp.float32)
            return 0

        lax.fori_loop(0, CH, zrow, 0)

        rows_per_tile = NPAD // NS

        def zcopy(r, _):
            pltpu.sync_copy(zbuf, acc_sh.at[pl.ds(s * rows_per_tile + r * CH, CH)])
            return 0

        lax.fori_loop(0, rows_per_tile // CH, zcopy, 0)
        plsc.subcore_barrier()

        def edge_body(g, _):
            base = w * PT + g * CH
            pltpu.sync_copy(src_hbm.at[pl.ds(base, CH)], sbuf)
            pltpu.sync_copy(dst_hbm.at[pl.ds(base, CH)], dbuf)
            pltpu.async_copy(z_hbm.at[sbuf], rows, sem).wait()
            pltpu.sync_copy(rows, acc_sh.at[dbuf], add=True)
            return 0

        lax.fori_loop(0, PT // CH, edge_body, 0)
        plsc.subcore_barrier()

        pltpu.sync_copy(
            acc_sh.at[pl.ds(s * rows_per_tile, rows_per_tile)],
            out_hbm.at[c, pl.ds(s * rows_per_tile, rows_per_tile)],
        )

    return k(z, srcp, dstp)


_BR = 1024


def _tc_first(xp, W1, deg2d):
    def body(x_ref, w_ref, deg_ref, z_ref, dinv_ref):
        dinv = lax.rsqrt(deg_ref[...] + 1.0)
        dinv_ref[...] = dinv
        z_ref[...] = (
            jnp.dot(x_ref[...], w_ref[...], preferred_element_type=jnp.float32)
            * dinv
        )

    return pl.pallas_call(
        body,
        grid=(NPAD // _BR,),
        in_specs=[
            pl.BlockSpec((_BR, D), lambda i: (i, 0)),
            pl.BlockSpec((D, D), lambda i: (0, 0)),
            pl.BlockSpec((_BR, 1), lambda i: (i, 0)),
        ],
        out_specs=[
            pl.BlockSpec((_BR, D), lambda i: (i, 0)),
            pl.BlockSpec((_BR, 1), lambda i: (i, 0)),
        ],
        out_shape=[
            jax.ShapeDtypeStruct((NPAD, D), jnp.float32),
            jax.ShapeDtypeStruct((NPAD, 1), jnp.float32),
        ],
    )(xp, W1, deg2d)


def _tc_mid(agg_a, agg_b, z1, dinv, b1, W2):
    def body(a_ref, bb_ref, z_ref, dinv_ref, bias_ref, w_ref, out_ref):
        h = (a_ref[...] + bb_ref[...] + z_ref[...]) * dinv_ref[...] + bias_ref[...]
        h = jnp.maximum(h, 0.0)
        out_ref[...] = (
            jnp.dot(h, w_ref[...], preferred_element_type=jnp.float32)
            * dinv_ref[...]
        )

    return pl.pallas_call(
        body,
        grid=(NPAD // _BR,),
        in_specs=[
            pl.BlockSpec((_BR, D), lambda i: (i, 0)),
            pl.BlockSpec((_BR, D), lambda i: (i, 0)),
            pl.BlockSpec((_BR, D), lambda i: (i, 0)),
            pl.BlockSpec((_BR, 1), lambda i: (i, 0)),
            pl.BlockSpec((1, D), lambda i: (0, 0)),
            pl.BlockSpec((D, D), lambda i: (0, 0)),
        ],
        out_specs=pl.BlockSpec((_BR, D), lambda i: (i, 0)),
        out_shape=jax.ShapeDtypeStruct((NPAD, D), jnp.float32),
    )(agg_a, agg_b, z1, dinv, b1, W2)


def _tc_last(agg_a, agg_b, z2, dinv, b2):
    def body(a_ref, bb_ref, z_ref, dinv_ref, bias_ref, out_ref):
        out_ref[...] = (
            (a_ref[...] + bb_ref[...] + z_ref[...]) * dinv_ref[...]
            + bias_ref[...]
        )

    return pl.pallas_call(
        body,
        grid=(NPAD // _BR,),
        in_specs=[
            pl.BlockSpec((_BR, D), lambda i: (i, 0)),
            pl.BlockSpec((_BR, D), lambda i: (i, 0)),
            pl.BlockSpec((_BR, D), lambda i: (i, 0)),
            pl.BlockSpec((_BR, 1), lambda i: (i, 0)),
            pl.BlockSpec((1, D), lambda i: (0, 0)),
        ],
        out_specs=pl.BlockSpec((_BR, D), lambda i: (i, 0)),
        out_shape=jax.ShapeDtypeStruct((NPAD, D), jnp.float32),
    )(agg_a, agg_b, z2, dinv, b2)


def kernel(node_features, edges, W1, b1, W2, b2):
    src = edges[:, 0].astype(jnp.int32)
    dst = edges[:, 1].astype(jnp.int32)
    pad = EPAD - E
    srcp = jnp.concatenate([src, jnp.zeros((pad,), jnp.int32)])
    dstp = jnp.concatenate([dst, jnp.full((pad,), N, jnp.int32)])

    xp = jnp.pad(node_features, ((0, NPAD - N), (0, 0)))
    b1r = b1.reshape(1, D)
    b2r = b2.reshape(1, D)

    deg_p = _deg_kernel(dstp)
    deg2d = (deg_p[0, :, 0] + deg_p[1, :, 0]).reshape(NPAD, 1)
    z1, dinv = _tc_first(xp, W1, deg2d)
    agg1 = _scatter_kernel(z1, srcp, dstp)
    z2 = _tc_mid(agg1[0], agg1[1], z1, dinv, b1r, W2)
    agg2 = _scatter_kernel(z2, srcp, dstp)
    out = _tc_last(agg2[0], agg2[1], z2, dinv, b2r)
    return out[:N]
